# Initial kernel scaffold; baseline (speedup 1.0000x reference)
import jax
import jax.numpy as jnp
from jax.experimental import pallas as pl


def kernel(pos, embedding):
    raise NotImplementedError("write your pallas kernel here")


# speedup vs baseline: 4.7266x; 4.7266x over previous
import functools

import jax
import jax.numpy as jnp
import numpy as np
from jax import lax
from jax.experimental import pallas as pl
from jax.experimental.pallas import tpu as pltpu
from jax.experimental.pallas import tpu_sc as plsc

_SCALE = 3.0
_LANES = 16
_NC = 2
_NS = 16
_NW = _NC * _NS


def _make_sc_lookup(B, V, D, dx):
    Bw = B // _NW
    C = 1024
    K = C // 128
    n_chunks = Bw // C

    mesh = plsc.VectorSubcoreMesh(core_axis_name="c", subcore_axis_name="s")
    mul = float(np.float32(dx) * (np.float32(1.0) / np.float32(_SCALE)))
    add = float(dx)
    hi = float(V - 1)

    @functools.partial(
        pl.kernel,
        mesh=mesh,
        out_type=jax.ShapeDtypeStruct((B, D), jnp.float32),
        scratch_types=[
            pltpu.VMEM((C,), jnp.float32),
            pltpu.VMEM((C,), jnp.int32),
            pltpu.VMEM((C, D), jnp.float32),
            pltpu.SemaphoreType.DMA,
        ],
        compiler_params=pltpu.CompilerParams(use_tc_tiling_on_sc=False),
    )
    def lookup(pos_hbm, tab_hbm, out_hbm, pos_v, idx_v, rows_v, sem):
        wid = lax.axis_index("s") * _NC + lax.axis_index("c")
        base = wid * Bw

        def chunk(ci, carry):
            off = base + ci * C
            pltpu.sync_copy(pos_hbm.at[pl.ds(off, C)], pos_v)

            def quant(i, c2):
                p = pos_v[pl.ds(i * _LANES, _LANES)]
                t = p * mul
                t = t + add
                t = jnp.minimum(jnp.maximum(t, 0.0), hi)
                idx_v[pl.ds(i * _LANES, _LANES)] = t.astype(jnp.int32)
                return c2

            lax.fori_loop(0, C // _LANES, quant, 0)

            copies = [
                pltpu.async_copy(
                    tab_hbm.at[idx_v.at[pl.ds(j * 128, 128)]],
                    rows_v.at[pl.ds(j * 128, 128)],
                    sem,
                )
                for j in range(K)
            ]
            for cp in copies:
                cp.wait()
            pltpu.sync_copy(rows_v, out_hbm.at[pl.ds(off, C)])
            return carry

        lax.fori_loop(0, n_chunks, chunk, 0)

    return lookup


def kernel(pos, embedding):
    B0, B1 = pos.shape
    V, D = embedding.shape
    B = B0 * B1
    dx = (V - 1) // 2
    pos_flat = pos.reshape(B)
    out = _make_sc_lookup(B, V, D, dx)(pos_flat, embedding)
    return out.reshape(B0, B1, D)


# speedup vs baseline: 4.9491x; 1.0471x over previous
import functools

import jax
import jax.numpy as jnp
import numpy as np
from jax import lax
from jax.experimental import pallas as pl
from jax.experimental.pallas import tpu as pltpu
from jax.experimental.pallas import tpu_sc as plsc

_SCALE = 3.0
_LANES = 16
_NC = 2
_NS = 16
_NW = _NC * _NS


def _make_sc_lookup(B, V, D, dx):
    Bw = B // _NW
    C = 1024
    K = C // 128
    n_chunks = Bw // C
    assert n_chunks >= 2 and n_chunks % 2 == 0

    mesh = plsc.VectorSubcoreMesh(core_axis_name="c", subcore_axis_name="s")
    mul = float(np.float32(dx) * (np.float32(1.0) / np.float32(_SCALE)))
    add = float(dx)
    hi = float(V - 1)

    @functools.partial(
        pl.kernel,
        mesh=mesh,
        out_type=jax.ShapeDtypeStruct((B, D), jnp.float32),
        scratch_types=[
            pltpu.VMEM((2, C), jnp.float32),
            pltpu.VMEM((2, C), jnp.int32),
            pltpu.VMEM((2, C, D), jnp.float32),
            pltpu.SemaphoreType.DMA,
            pltpu.SemaphoreType.DMA,
            pltpu.SemaphoreType.DMA,
            pltpu.SemaphoreType.DMA,
            pltpu.SemaphoreType.DMA,
            pltpu.SemaphoreType.DMA,
        ],
        compiler_params=pltpu.CompilerParams(use_tc_tiling_on_sc=False),
    )
    def lookup(pos_hbm, tab_hbm, out_hbm, pos_v, idx_v, rows_v,
               sp0, sp1, sg0, sg1, sw0, sw1):
        wid = lax.axis_index("s") * _NC + lax.axis_index("c")
        base = wid * Bw
        sp, sg, sw = (sp0, sp1), (sg0, sg1), (sw0, sw1)

        def pos_cp(gi, b):
            return pltpu.make_async_copy(
                pos_hbm.at[pl.ds(base + gi * C, C)], pos_v.at[b], sp[b])

        def gat_cp(b, j):
            return pltpu.make_async_copy(
                tab_hbm.at[idx_v.at[b, pl.ds(j * 128, 128)]],
                rows_v.at[b, pl.ds(j * 128, 128)],
                sg[b])

        def out_cp(gi, b):
            return pltpu.make_async_copy(
                rows_v.at[b], out_hbm.at[pl.ds(base + gi * C, C)], sw[b])

        def quantize(b):
            def qi(i, c2):
                p = pos_v[b, pl.ds(i * _LANES, _LANES)]
                t = p * mul
                t = t + add
                t = jnp.minimum(jnp.maximum(t, 0.0), hi)
                idx_v[b, pl.ds(i * _LANES, _LANES)] = t.astype(jnp.int32)
                return c2
            lax.fori_loop(0, C // _LANES, qi, 0)

        pos_cp(0, 0).start()
        pos_cp(1, 1).start()

        def outer(g, carry):
            for b in range(2):
                gi = g * 2 + b
                pos_cp(gi, b).wait()
                quantize(b)

                @pl.when(gi + 2 < n_chunks)
                def _():
                    pos_cp(gi + 2, b).start()

                @pl.when(gi >= 1)
                def _():
                    for j in range(K):
                        gat_cp(1 - b, j).wait()
                    out_cp(gi - 1, 1 - b).start()

                @pl.when(gi >= 2)
                def _():
                    out_cp(gi - 2, b).wait()

                for j in range(K):
                    gat_cp(b, j).start()
            return carry

        lax.fori_loop(0, n_chunks // 2, outer, 0)

        lb = (n_chunks - 1) % 2
        for j in range(K):
            gat_cp(lb, j).wait()
        out_cp(n_chunks - 1, lb).start()
        out_cp(n_chunks - 2, 1 - lb).wait()
        out_cp(n_chunks - 1, lb).wait()

    return lookup


def kernel(pos, embedding):
    B0, B1 = pos.shape
    V, D = embedding.shape
    B = B0 * B1
    dx = (V - 1) // 2
    pos_flat = pos.reshape(B)
    out = _make_sc_lookup(B, V, D, dx)(pos_flat, embedding)
    return out.reshape(B0, B1, D)
